# baseline (speedup 1.0000x reference)
import math

import jax
import jax.numpy as jnp
from jax import lax
from jax.experimental import pallas as pl
from jax.experimental.pallas import tpu as pltpu
from jax.experimental.pallas import tpu_sc as plsc

F32 = jnp.float32
I32 = jnp.int32

B_, T_, N_, E_ = 4, 8, 10000, 160000
G_ = B_ * T_
F_, H_, C_ = 128, 128, 10
EPS_ = 1e-5
BNC = 1.0 / math.sqrt(1.0 + EPS_)

NPAD = 10240
CH = 128
CPT = 80
TILES = 16
GPC = 16
EPAD = TILES * CPT * CH
ROWS_PT = NPAD // TILES

BR = 1024
NB = NPAD // BR

def _mesh():
    return plsc.VectorSubcoreMesh(core_axis_name="c", subcore_axis_name="s")


def _deg_body(dst_hbm, deg_hbm, dgb, ones, zbuf, deg_sh, sem):
    cid = lax.axis_index("c")
    sid = lax.axis_index("s")
    row0 = sid * ROWS_PT

    def _init_ones(i, carry):
        for j in range(8):
            ones[i, pl.ds(j * 16, 16)] = jnp.full((16,), 1.0, F32)
        return carry

    lax.fori_loop(0, CH, _init_ones, 0)

    def _init_z(i, carry):
        for j in range(8):
            zbuf[i, pl.ds(j * 16, 16)] = jnp.zeros((16,), F32)
        return carry

    lax.fori_loop(0, 16, _init_z, 0)

    def _per_graph(gi, carry):
        g_id = cid * GPC + gi

        def _zero(k, c2):
            pltpu.sync_copy(zbuf, deg_sh.at[pl.ds(row0 + k * 16, 16)])
            return c2

        lax.fori_loop(0, ROWS_PT // 16, _zero, 0)
        plsc.subcore_barrier()

        def _grp(k, c2):
            pltpu.sync_copy(dst_hbm.at[g_id, sid, pl.ds(k * 8, 8)], dgb)
            descs = [
                pltpu.async_copy(ones, deg_sh.at[dgb.at[j]], sem, add=True)
                for j in range(8)
            ]
            for d in descs:
                d.wait()
            return c2

        lax.fori_loop(0, CPT // 8, _grp, 0)
        plsc.subcore_barrier()
        pltpu.sync_copy(deg_sh.at[pl.ds(row0, ROWS_PT)],
                        deg_hbm.at[g_id, pl.ds(row0, ROWS_PT)])
        return carry

    lax.fori_loop(0, GPC, _per_graph, 0)


def _compute_deg(dst_r):
    return pl.kernel(
        _deg_body,
        out_type=jax.ShapeDtypeStruct((G_, NPAD, H_), F32),
        mesh=_mesh(),
        scratch_types=[
            pltpu.VMEM((8, CH), I32),
            pltpu.VMEM((CH, H_), F32),
            pltpu.VMEM((16, H_), F32),
            pltpu.VMEM_SHARED((NPAD, H_), F32),
            pltpu.SemaphoreType.DMA,
        ],
    )(dst_r)


def _scatter_body(eg_hbm, gt_hbm, y_hbm, egb, rows, zbuf, tidx, y_sh,
                  gsem, ssem, isem):
    cid = lax.axis_index("c")
    sid = lax.axis_index("s")
    row0 = sid * ROWS_PT

    def _init_z(i, carry):
        for j in range(8):
            zbuf[i, pl.ds(j * 16, 16)] = jnp.zeros((16,), F32)
        return carry

    lax.fori_loop(0, 16, _init_z, 0)
    for j in range(8):
        tidx[0, pl.ds(j * 16, 16)] = jnp.full((16,), N_, I32)

    def _drain_row(jj, sem):
        pltpu.make_async_copy(gt_hbm.at[pl.ds(0, CH)], rows.at[jj],
                              sem).wait()

    def _drain_idx():
        pltpu.make_async_copy(eg_hbm.at[0, 0, 0], egb.at[0], isem).wait()

    def _gather(h, j, buf):
        return pltpu.async_copy(gt_hbm.at[egb.at[h, 0, j]], rows.at[buf],
                                gsem)

    def _scat(h, j, buf):
        return pltpu.async_copy(rows.at[buf], y_sh.at[egb.at[h, 1, j]],
                                ssem, add=True)

    def _steps(g_id, k, prefetch):
        for j in range(8):
            h, jj = j // 4, j % 4
            buf = j % 2
            _drain_row(buf, gsem)
            _drain_row(1 - buf, ssem)
            if j == 0:
                pltpu.async_copy(eg_hbm.at[g_id, sid, 2 * k + 1],
                                 egb.at[1], isem)
            if j == 4 and prefetch:
                pltpu.async_copy(eg_hbm.at[g_id, sid, 2 * k + 2],
                                 egb.at[0], isem)
            _scat(h, jj, buf)
            if j < 7:
                if j == 3:
                    _drain_idx()
                _gather((j + 1) // 4, (j + 1) % 4, 1 - buf)
            elif prefetch:
                _drain_idx()
                _gather(0, 0, 1 - buf)

    def _per_graph(gi, carry):
        g_id = cid * GPC + gi

        def _zero(k, c2):
            pltpu.sync_copy(zbuf, y_sh.at[pl.ds(row0 + k * 16, 16)])
            return c2

        lax.fori_loop(0, ROWS_PT // 16, _zero, 0)
        plsc.subcore_barrier()

        pltpu.async_copy(eg_hbm.at[g_id, sid, 0], egb.at[0], isem).wait()
        pltpu.async_copy(rows.at[1], y_sh.at[tidx.at[0]], ssem, add=True)
        _gather(0, 0, 0)

        def _body(k, c2):
            _steps(g_id, k, True)
            return c2

        lax.fori_loop(0, CPT // 8 - 1, _body, 0)
        _steps(g_id, CPT // 8 - 1, False)
        _drain_row(1, ssem)
        plsc.subcore_barrier()
        pltpu.sync_copy(y_sh.at[pl.ds(row0, ROWS_PT)],
                        y_hbm.at[g_id, pl.ds(row0, ROWS_PT)])
        return carry

    lax.fori_loop(0, GPC, _per_graph, 0)


def _sc_scatter(eg_r, g_flat):
    return pl.kernel(
        _scatter_body,
        out_type=jax.ShapeDtypeStruct((G_, NPAD, H_), F32),
        mesh=_mesh(),
        scratch_types=[
            pltpu.VMEM((2, 2, 4, CH), I32),
            pltpu.VMEM((2, CH, H_), F32),
            pltpu.VMEM((16, H_), F32),
            pltpu.VMEM((1, CH), I32),
            pltpu.VMEM_SHARED((NPAD, H_), F32),
            pltpu.SemaphoreType.DMA,
            pltpu.SemaphoreType.DMA,
            pltpu.SemaphoreType.DMA,
        ],
    )(eg_r, g_flat)


def _p1_body(x_ref, w_ref, deg_ref, g_ref):
    dinv = lax.rsqrt(deg_ref[...][:, 0:1] + 1.0)
    g_ref[...] = dinv * jnp.dot(x_ref[...], w_ref[...],
                                preferred_element_type=F32)


def _p31_body(y_ref, g_ref, h_ref, deg_ref, b_ref, gam_ref, bet_ref, wn_ref,
              ho_ref, go_ref):
    r = pl.program_id(1)
    dinv = lax.rsqrt(deg_ref[...][:, 0:1] + 1.0)
    t = dinv * (y_ref[...] + g_ref[...]) + b_ref[...]
    t = t * (gam_ref[...] * BNC) + bet_ref[...]
    hn = jnp.maximum(t, 0.0)
    rows = r * BR + lax.broadcasted_iota(I32, (BR, 1), 0)
    hp = jnp.where(rows < N_, hn + h_ref[...], 0.0)
    ho_ref[...] = hp
    go_ref[...] = dinv * jnp.dot(hp, wn_ref[...], preferred_element_type=F32)


def _p3f_body(y_ref, g_ref, h_ref, deg_ref, b_ref, gam_ref, bet_ref, o_ref):
    r = pl.program_id(1)
    dinv = lax.rsqrt(deg_ref[...][:, 0:1] + 1.0)
    t = dinv * (y_ref[...] + g_ref[...]) + b_ref[...]
    t = t * (gam_ref[...] * BNC) + bet_ref[...]
    hn = jnp.maximum(t, 0.0)
    rows = r * BR + lax.broadcasted_iota(I32, (BR, 1), 0)
    hp = jnp.where(rows < N_, hn + h_ref[...], 0.0)

    @pl.when(r == 0)
    def _():
        o_ref[...] = jnp.zeros_like(o_ref)

    o_ref[...] += jnp.sum(hp, axis=0, keepdims=True)


_BS_BIG = pl.BlockSpec((None, BR, H_), lambda g, r: (g, r, 0))
_BS_DEG = pl.BlockSpec((None, BR, H_), lambda g, r: (g, r, 0))
_BS_VEC = pl.BlockSpec((1, H_), lambda g, r: (0, 0))
_BS_W = pl.BlockSpec((H_, H_), lambda g, r: (0, 0))
_TC_PARAMS = pltpu.CompilerParams(
    dimension_semantics=("parallel", "arbitrary"))


def _p1(x, w, deg):
    return pl.pallas_call(
        _p1_body,
        grid=(G_, NB),
        in_specs=[_BS_BIG, _BS_W, _BS_DEG],
        out_specs=_BS_BIG,
        out_shape=jax.ShapeDtypeStruct((G_, NPAD, H_), F32),
        compiler_params=_TC_PARAMS,
    )(x, w, deg)


def _p31(y, g, h, deg, b, gam, bet, wn):
    return pl.pallas_call(
        _p31_body,
        grid=(G_, NB),
        in_specs=[_BS_BIG, _BS_BIG, _BS_BIG, _BS_DEG,
                  _BS_VEC, _BS_VEC, _BS_VEC, _BS_W],
        out_specs=[_BS_BIG, _BS_BIG],
        out_shape=[jax.ShapeDtypeStruct((G_, NPAD, H_), F32)] * 2,
        compiler_params=_TC_PARAMS,
    )(y, g, h, deg, b, gam, bet, wn)


def _p3f(y, g, h, deg, b, gam, bet):
    return pl.pallas_call(
        _p3f_body,
        grid=(G_, NB),
        in_specs=[_BS_BIG, _BS_BIG, _BS_BIG, _BS_DEG,
                  _BS_VEC, _BS_VEC, _BS_VEC],
        out_specs=pl.BlockSpec((None, 1, H_), lambda g, r: (g, 0, 0)),
        out_shape=jax.ShapeDtypeStruct((G_, 1, H_), F32),
        compiler_params=_TC_PARAMS,
    )(y, g, h, deg, b, gam, bet)


def _head_body(enc_ref,
               wi00, wh00, bi00, bh00, wi01, wh01, bi01, bh01,
               wi10, wh10, bi10, bh10, wi11, wh11, bi11, bh11,
               aw1, ab1, aw2, ab2, f1w, f1b, f2w, f2b, out_ref):
    enc = enc_ref[...]

    def run_dir(xs, wi, wh, bi, bh):
        h = jnp.zeros((B_, H_), F32)
        c = jnp.zeros((B_, H_), F32)
        hs = []
        for t in range(T_):
            gates = (jnp.dot(xs[t], wi[...], preferred_element_type=F32)
                     + jnp.dot(h, wh[...], preferred_element_type=F32)
                     + bi[...] + bh[...])
            i = jax.nn.sigmoid(gates[:, 0:H_])
            f = jax.nn.sigmoid(gates[:, H_:2 * H_])
            gg = jnp.tanh(gates[:, 2 * H_:3 * H_])
            o = jax.nn.sigmoid(gates[:, 3 * H_:4 * H_])
            c = f * c + i * gg
            h = o * jnp.tanh(c)
            hs.append(h)
        return hs

    xs0 = [enc[t] for t in range(T_)]
    hf = run_dir(xs0, wi00, wh00, bi00, bh00)
    hb = run_dir(xs0[::-1], wi01, wh01, bi01, bh01)[::-1]
    xs1 = [jnp.concatenate([hf[t], hb[t]], axis=1) for t in range(T_)]
    hf2 = run_dir(xs1, wi10, wh10, bi10, bh10)
    hb2 = run_dir(xs1[::-1], wi11, wh11, bi11, bh11)[::-1]
    outs = [jnp.concatenate([hf2[t], hb2[t]], axis=1) for t in range(T_)]

    scores = [
        jnp.dot(jnp.tanh(jnp.dot(o_, aw1[...], preferred_element_type=F32)
                         + ab1[...]),
                aw2[...], preferred_element_type=F32) + ab2[...]
        for o_ in outs
    ]
    m = scores[0]
    for s_ in scores[1:]:
        m = jnp.maximum(m, s_)
    es = [jnp.exp(s_ - m) for s_ in scores]
    den = es[0]
    for e_ in es[1:]:
        den = den + e_
    ws = outs[0] * (es[0] / den)
    for t in range(1, T_):
        ws = ws + outs[t] * (es[t] / den)

    hfc = jnp.maximum(
        jnp.dot(ws, f1w[...], preferred_element_type=F32) + f1b[...], 0.0)
    logits = jnp.dot(hfc, f2w[...], preferred_element_type=F32) + f2b[...]
    mx = jnp.max(logits, axis=1, keepdims=True)
    lse = jnp.log(jnp.sum(jnp.exp(logits - mx), axis=1, keepdims=True)) + mx
    out_ref[...] = logits - lse


def _head(enc_t, *args):
    return pl.pallas_call(
        _head_body,
        out_shape=jax.ShapeDtypeStruct((B_, C_), F32),
    )(enc_t, *args)


def kernel(x, edge_index, params):
    xg = x.reshape(G_, N_, F_)
    xp = jnp.pad(xg, ((0, 0), (0, NPAD - N_), (0, 0)))

    ei = edge_index.reshape(G_, 2, E_)
    src = jnp.pad(ei[:, 0], ((0, 0), (0, EPAD - E_)), constant_values=N_)
    dst = jnp.pad(ei[:, 1], ((0, 0), (0, EPAD - E_)), constant_values=N_)
    srca = src + (jnp.arange(G_, dtype=I32) * NPAD)[:, None]
    dst_r = dst.reshape(G_, TILES, CPT, CH)
    eg_r = jnp.stack(
        [srca.reshape(G_, TILES, CPT // 4, 4, CH),
         dst.reshape(G_, TILES, CPT // 4, 4, CH)],
        axis=3)

    deg = _compute_deg(dst_r)

    h = xp
    g = _p1(xp, params['gnn_W'][0], deg)
    sums = None
    for i in range(3):
        y = _sc_scatter(eg_r, g.reshape(G_ * NPAD, H_))
        b = params['gnn_b'][i].reshape(1, H_)
        gam = params['bn_gamma'][i].reshape(1, H_)
        bet = params['bn_beta'][i].reshape(1, H_)
        if i < 2:
            h, g = _p31(y, g, h, deg, b, gam, bet, params['gnn_W'][i + 1])
        else:
            sums = _p3f(y, g, h, deg, b, gam, bet)

    enc = (sums[:, 0, :] / float(N_)).reshape(B_, T_, H_)
    enc_t = jnp.transpose(enc, (1, 0, 2))

    head_args = []
    for layer in range(2):
        for d in range(2):
            p = params['lstm'][layer][d]
            head_args += [p['Wih'].T, p['Whh'].T,
                          p['bih'].reshape(1, 4 * H_),
                          p['bhh'].reshape(1, 4 * H_)]
    head_args += [params['att_W1'], params['att_b1'].reshape(1, H_),
                  params['att_W2'], params['att_b2'].reshape(1, 1),
                  params['fc1_W'], params['fc1_b'].reshape(1, H_),
                  params['fc2_W'], params['fc2_b'].reshape(1, C_)]

    return _head(enc_t, *head_args)


# speedup vs baseline: 1.0900x; 1.0900x over previous
import math

import jax
import jax.numpy as jnp
from jax import lax
from jax.experimental import pallas as pl
from jax.experimental.pallas import tpu as pltpu
from jax.experimental.pallas import tpu_sc as plsc

F32 = jnp.float32
I32 = jnp.int32

B_, T_, N_, E_ = 4, 8, 10000, 160000
G_ = B_ * T_
F_, H_, C_ = 128, 128, 10
EPS_ = 1e-5
BNC = 1.0 / math.sqrt(1.0 + EPS_)

NPAD = 10240
CH = 128
CPT = 80
TILES = 16
GPC = 16
EPAD = TILES * CPT * CH
CR = 64
NGRP = EPAD // TILES // (8 * CR)
ROWS_PT = NPAD // TILES

BR = 1024
NB = NPAD // BR

def _mesh():
    return plsc.VectorSubcoreMesh(core_axis_name="c", subcore_axis_name="s")


def _deg_body(dst_hbm, deg_hbm, dgb, ones, zbuf, deg_sh, sem):
    cid = lax.axis_index("c")
    sid = lax.axis_index("s")
    row0 = sid * ROWS_PT

    def _init_ones(i, carry):
        for j in range(8):
            ones[i, pl.ds(j * 16, 16)] = jnp.full((16,), 1.0, F32)
        return carry

    lax.fori_loop(0, CH, _init_ones, 0)

    def _init_z(i, carry):
        for j in range(8):
            zbuf[i, pl.ds(j * 16, 16)] = jnp.zeros((16,), F32)
        return carry

    lax.fori_loop(0, 16, _init_z, 0)

    def _per_graph(gi, carry):
        g_id = cid * GPC + gi

        def _zero(k, c2):
            pltpu.sync_copy(zbuf, deg_sh.at[pl.ds(row0 + k * 16, 16)])
            return c2

        lax.fori_loop(0, ROWS_PT // 16, _zero, 0)
        plsc.subcore_barrier()

        def _grp(k, c2):
            pltpu.sync_copy(dst_hbm.at[g_id, sid, pl.ds(k * 8, 8)], dgb)
            descs = [
                pltpu.async_copy(ones, deg_sh.at[dgb.at[j]], sem, add=True)
                for j in range(8)
            ]
            for d in descs:
                d.wait()
            return c2

        lax.fori_loop(0, CPT // 8, _grp, 0)
        plsc.subcore_barrier()
        pltpu.sync_copy(deg_sh.at[pl.ds(row0, ROWS_PT)],
                        deg_hbm.at[g_id, pl.ds(row0, ROWS_PT)])
        return carry

    lax.fori_loop(0, GPC, _per_graph, 0)


def _compute_deg(dst_r):
    return pl.kernel(
        _deg_body,
        out_type=jax.ShapeDtypeStruct((G_, NPAD, H_), F32),
        mesh=_mesh(),
        scratch_types=[
            pltpu.VMEM((8, CH), I32),
            pltpu.VMEM((CH, H_), F32),
            pltpu.VMEM((16, H_), F32),
            pltpu.VMEM_SHARED((NPAD, H_), F32),
            pltpu.SemaphoreType.DMA,
        ],
    )(dst_r)


def _scatter_body(eg_hbm, gt_hbm, y_hbm, egb, rows, zbuf, tidx, y_sh,
                  gsem, ssem, isem):
    cid = lax.axis_index("c")
    sid = lax.axis_index("s")
    row0 = sid * ROWS_PT

    def _init_z(i, carry):
        for j in range(8):
            zbuf[i, pl.ds(j * 16, 16)] = jnp.zeros((16,), F32)
        return carry

    lax.fori_loop(0, 16, _init_z, 0)
    for j in range(4):
        tidx[0, pl.ds(j * 16, 16)] = jnp.full((16,), N_, I32)

    def _drain_row(jj, sem):
        pltpu.make_async_copy(gt_hbm.at[pl.ds(0, CR)], rows.at[jj],
                              sem).wait()

    def _drain_idx():
        pltpu.make_async_copy(eg_hbm.at[0, 0, 0], egb.at[0], isem).wait()

    def _gather(h, j, buf):
        return pltpu.async_copy(gt_hbm.at[egb.at[h, 0, j]], rows.at[buf],
                                gsem)

    def _scat(h, j, buf):
        return pltpu.async_copy(rows.at[buf], y_sh.at[egb.at[h, 1, j]],
                                ssem, add=True)

    def _steps(g_id, k, prefetch):
        for j in range(16):
            h, jj = j // 8, j % 8
            buf = j % 4
            _drain_row(buf, gsem)
            _drain_row((j + 3) % 4, ssem)
            _scat(h, jj, buf)
            if j == 0:
                pltpu.async_copy(eg_hbm.at[g_id, sid, 2 * k + 1],
                                 egb.at[1], isem)
            if j == 8 and prefetch:
                pltpu.async_copy(eg_hbm.at[g_id, sid, 2 * k + 2],
                                 egb.at[0], isem)
            nc = j + 3
            if j == 5:
                _drain_idx()
            if prefetch and j == 13:
                _drain_idx()
            if prefetch or nc < 16:
                _gather((nc // 8) % 2, nc % 8, nc % 4)

    def _per_graph(gi, carry):
        g_id = cid * GPC + gi

        def _zero(k, c2):
            pltpu.sync_copy(zbuf, y_sh.at[pl.ds(row0 + k * 16, 16)])
            return c2

        lax.fori_loop(0, ROWS_PT // 16, _zero, 0)
        plsc.subcore_barrier()

        pltpu.async_copy(eg_hbm.at[g_id, sid, 0], egb.at[0], isem).wait()
        pltpu.async_copy(rows.at[3], y_sh.at[tidx.at[0]], ssem, add=True)
        _gather(0, 0, 0)
        _gather(0, 1, 1)
        _gather(0, 2, 2)

        def _body(k, c2):
            _steps(g_id, k, True)
            return c2

        lax.fori_loop(0, NGRP // 2 - 1, _body, 0)
        _steps(g_id, NGRP // 2 - 1, False)
        _drain_row(3, ssem)
        plsc.subcore_barrier()
        pltpu.sync_copy(y_sh.at[pl.ds(row0, ROWS_PT)],
                        y_hbm.at[g_id, pl.ds(row0, ROWS_PT)])
        return carry

    lax.fori_loop(0, GPC, _per_graph, 0)


def _sc_scatter(eg_r, g_flat):
    return pl.kernel(
        _scatter_body,
        out_type=jax.ShapeDtypeStruct((G_, NPAD, H_), F32),
        mesh=_mesh(),
        scratch_types=[
            pltpu.VMEM((2, 2, 8, CR), I32),
            pltpu.VMEM((4, CR, H_), F32),
            pltpu.VMEM((16, H_), F32),
            pltpu.VMEM((1, CR), I32),
            pltpu.VMEM_SHARED((NPAD, H_), F32),
            pltpu.SemaphoreType.DMA,
            pltpu.SemaphoreType.DMA,
            pltpu.SemaphoreType.DMA,
        ],
    )(eg_r, g_flat)


def _p1_body(x_ref, w_ref, deg_ref, g_ref):
    dinv = lax.rsqrt(deg_ref[...][:, 0:1] + 1.0)
    g_ref[...] = dinv * jnp.dot(x_ref[...], w_ref[...],
                                preferred_element_type=F32)


def _p31_body(y_ref, g_ref, h_ref, deg_ref, b_ref, gam_ref, bet_ref, wn_ref,
              ho_ref, go_ref):
    r = pl.program_id(1)
    dinv = lax.rsqrt(deg_ref[...][:, 0:1] + 1.0)
    t = dinv * (y_ref[...] + g_ref[...]) + b_ref[...]
    t = t * (gam_ref[...] * BNC) + bet_ref[...]
    hn = jnp.maximum(t, 0.0)
    rows = r * BR + lax.broadcasted_iota(I32, (BR, 1), 0)
    hp = jnp.where(rows < N_, hn + h_ref[...], 0.0)
    ho_ref[...] = hp
    go_ref[...] = dinv * jnp.dot(hp, wn_ref[...], preferred_element_type=F32)


def _p3f_body(y_ref, g_ref, h_ref, deg_ref, b_ref, gam_ref, bet_ref, o_ref):
    r = pl.program_id(1)
    dinv = lax.rsqrt(deg_ref[...][:, 0:1] + 1.0)
    t = dinv * (y_ref[...] + g_ref[...]) + b_ref[...]
    t = t * (gam_ref[...] * BNC) + bet_ref[...]
    hn = jnp.maximum(t, 0.0)
    rows = r * BR + lax.broadcasted_iota(I32, (BR, 1), 0)
    hp = jnp.where(rows < N_, hn + h_ref[...], 0.0)

    @pl.when(r == 0)
    def _():
        o_ref[...] = jnp.zeros_like(o_ref)

    o_ref[...] += jnp.sum(hp, axis=0, keepdims=True)


_BS_BIG = pl.BlockSpec((None, BR, H_), lambda g, r: (g, r, 0))
_BS_DEG = pl.BlockSpec((None, BR, H_), lambda g, r: (g, r, 0))
_BS_VEC = pl.BlockSpec((1, H_), lambda g, r: (0, 0))
_BS_W = pl.BlockSpec((H_, H_), lambda g, r: (0, 0))
_TC_PARAMS = pltpu.CompilerParams(
    dimension_semantics=("parallel", "arbitrary"))


def _p1(x, w, deg):
    return pl.pallas_call(
        _p1_body,
        grid=(G_, NB),
        in_specs=[_BS_BIG, _BS_W, _BS_DEG],
        out_specs=_BS_BIG,
        out_shape=jax.ShapeDtypeStruct((G_, NPAD, H_), F32),
        compiler_params=_TC_PARAMS,
    )(x, w, deg)


def _p31(y, g, h, deg, b, gam, bet, wn):
    return pl.pallas_call(
        _p31_body,
        grid=(G_, NB),
        in_specs=[_BS_BIG, _BS_BIG, _BS_BIG, _BS_DEG,
                  _BS_VEC, _BS_VEC, _BS_VEC, _BS_W],
        out_specs=[_BS_BIG, _BS_BIG],
        out_shape=[jax.ShapeDtypeStruct((G_, NPAD, H_), F32)] * 2,
        compiler_params=_TC_PARAMS,
    )(y, g, h, deg, b, gam, bet, wn)


def _p3f(y, g, h, deg, b, gam, bet):
    return pl.pallas_call(
        _p3f_body,
        grid=(G_, NB),
        in_specs=[_BS_BIG, _BS_BIG, _BS_BIG, _BS_DEG,
                  _BS_VEC, _BS_VEC, _BS_VEC],
        out_specs=pl.BlockSpec((None, 1, H_), lambda g, r: (g, 0, 0)),
        out_shape=jax.ShapeDtypeStruct((G_, 1, H_), F32),
        compiler_params=_TC_PARAMS,
    )(y, g, h, deg, b, gam, bet)


def _head_body(enc_ref,
               wi00, wh00, bi00, bh00, wi01, wh01, bi01, bh01,
               wi10, wh10, bi10, bh10, wi11, wh11, bi11, bh11,
               aw1, ab1, aw2, ab2, f1w, f1b, f2w, f2b, out_ref):
    enc = enc_ref[...]

    def run_dir(xs, wi, wh, bi, bh):
        h = jnp.zeros((B_, H_), F32)
        c = jnp.zeros((B_, H_), F32)
        hs = []
        for t in range(T_):
            gates = (jnp.dot(xs[t], wi[...], preferred_element_type=F32)
                     + jnp.dot(h, wh[...], preferred_element_type=F32)
                     + bi[...] + bh[...])
            i = jax.nn.sigmoid(gates[:, 0:H_])
            f = jax.nn.sigmoid(gates[:, H_:2 * H_])
            gg = jnp.tanh(gates[:, 2 * H_:3 * H_])
            o = jax.nn.sigmoid(gates[:, 3 * H_:4 * H_])
            c = f * c + i * gg
            h = o * jnp.tanh(c)
            hs.append(h)
        return hs

    xs0 = [enc[t] for t in range(T_)]
    hf = run_dir(xs0, wi00, wh00, bi00, bh00)
    hb = run_dir(xs0[::-1], wi01, wh01, bi01, bh01)[::-1]
    xs1 = [jnp.concatenate([hf[t], hb[t]], axis=1) for t in range(T_)]
    hf2 = run_dir(xs1, wi10, wh10, bi10, bh10)
    hb2 = run_dir(xs1[::-1], wi11, wh11, bi11, bh11)[::-1]
    outs = [jnp.concatenate([hf2[t], hb2[t]], axis=1) for t in range(T_)]

    scores = [
        jnp.dot(jnp.tanh(jnp.dot(o_, aw1[...], preferred_element_type=F32)
                         + ab1[...]),
                aw2[...], preferred_element_type=F32) + ab2[...]
        for o_ in outs
    ]
    m = scores[0]
    for s_ in scores[1:]:
        m = jnp.maximum(m, s_)
    es = [jnp.exp(s_ - m) for s_ in scores]
    den = es[0]
    for e_ in es[1:]:
        den = den + e_
    ws = outs[0] * (es[0] / den)
    for t in range(1, T_):
        ws = ws + outs[t] * (es[t] / den)

    hfc = jnp.maximum(
        jnp.dot(ws, f1w[...], preferred_element_type=F32) + f1b[...], 0.0)
    logits = jnp.dot(hfc, f2w[...], preferred_element_type=F32) + f2b[...]
    mx = jnp.max(logits, axis=1, keepdims=True)
    lse = jnp.log(jnp.sum(jnp.exp(logits - mx), axis=1, keepdims=True)) + mx
    out_ref[...] = logits - lse


def _head(enc_t, *args):
    return pl.pallas_call(
        _head_body,
        out_shape=jax.ShapeDtypeStruct((B_, C_), F32),
    )(enc_t, *args)


def kernel(x, edge_index, params):
    xg = x.reshape(G_, N_, F_)
    xp = jnp.pad(xg, ((0, 0), (0, NPAD - N_), (0, 0)))

    ei = edge_index.reshape(G_, 2, E_)
    src = jnp.pad(ei[:, 0], ((0, 0), (0, EPAD - E_)), constant_values=N_)
    dst = jnp.pad(ei[:, 1], ((0, 0), (0, EPAD - E_)), constant_values=N_)
    srca = src + (jnp.arange(G_, dtype=I32) * NPAD)[:, None]
    dst_r = dst.reshape(G_, TILES, CPT, CH)
    eg_r = jnp.stack(
        [srca.reshape(G_, TILES, NGRP, 8, CR),
         dst.reshape(G_, TILES, NGRP, 8, CR)],
        axis=3)

    deg = _compute_deg(dst_r)

    h = xp
    g = _p1(xp, params['gnn_W'][0], deg)
    sums = None
    for i in range(3):
        y = _sc_scatter(eg_r, g.reshape(G_ * NPAD, H_))
        b = params['gnn_b'][i].reshape(1, H_)
        gam = params['bn_gamma'][i].reshape(1, H_)
        bet = params['bn_beta'][i].reshape(1, H_)
        if i < 2:
            h, g = _p31(y, g, h, deg, b, gam, bet, params['gnn_W'][i + 1])
        else:
            sums = _p3f(y, g, h, deg, b, gam, bet)

    enc = (sums[:, 0, :] / float(N_)).reshape(B_, T_, H_)
    enc_t = jnp.transpose(enc, (1, 0, 2))

    head_args = []
    for layer in range(2):
        for d in range(2):
            p = params['lstm'][layer][d]
            head_args += [p['Wih'].T, p['Whh'].T,
                          p['bih'].reshape(1, 4 * H_),
                          p['bhh'].reshape(1, 4 * H_)]
    head_args += [params['att_W1'], params['att_b1'].reshape(1, H_),
                  params['att_W2'], params['att_b2'].reshape(1, 1),
                  params['fc1_W'], params['fc1_b'].reshape(1, H_),
                  params['fc2_W'], params['fc2_b'].reshape(1, C_)]

    return _head(enc_t, *head_args)
